# baseline (speedup 1.0000x reference)
import functools

import jax
import jax.numpy as jnp
from jax import lax
from jax.experimental import pallas as pl
from jax.experimental.pallas import tpu as pltpu
from jax.experimental.pallas import tpu_sc as plsc

N = 10000
E = 320000
D = 128
DE = 16
H = 4
C = 32
HC = H * C
DEN = 16

NC = 2
NS = 16
NW = NC * NS
EPW = E // NW
NP = 10240
RPT = NP // NS
ZR = 16

KG = 40
GG = 5
EG = GG * KG
NBG = EPW // (2 * EG)
IRWG = E // KG // NW

KS = 80
NCH = EPW // KS
PAIRS = NCH // 2

RB = 2000
EB = 4000
PK = HC // 2


def _perm_feature(j):
    return jnp.where(j < PK, 2 * j, 2 * j - (HC - 1))


def _hsel_p():
    j = lax.broadcasted_iota(jnp.int32, (HC, H), 0)
    h = lax.broadcasted_iota(jnp.int32, (HC, H), 1)
    return (_perm_feature(j) // C == h).astype(jnp.float32)


def _pack_bf16(v):
    a = lax.bitcast_convert_type(v[:, :PK], jnp.int32)
    b = lax.bitcast_convert_type(v[:, PK:], jnp.int32)
    ra = lax.shift_right_logical(
        a + 0x7FFF + (lax.shift_right_logical(a, 16) & 1), 16)
    rb = lax.shift_right_logical(
        b + 0x7FFF + (lax.shift_right_logical(b, 16) & 1), 16)
    return lax.bitcast_convert_type(ra | lax.shift_left(rb, 16), jnp.float32)


def _unpack_bf16_halves(p):
    u = lax.bitcast_convert_type(p, jnp.int32)
    lo = lax.bitcast_convert_type(lax.shift_left(u, 16), jnp.float32)
    hi = lax.bitcast_convert_type(u & jnp.int32(-65536), jnp.float32)
    return lo, hi


def _proj_body(x_ref, wl_ref, bl_ref, wr_ref, br_ref,
               xl_ref, xr_ref, xlp_ref, xrp_ref):
    x = x_ref[...]
    xl = jnp.dot(x, wl_ref[...], preferred_element_type=jnp.float32) + bl_ref[...]
    xr = jnp.dot(x, wr_ref[...], preferred_element_type=jnp.float32) + br_ref[...]
    xl_ref[...] = xl
    xr_ref[...] = xr
    xlp_ref[...] = _pack_bf16(xl)
    xrp_ref[...] = _pack_bf16(xr)


def _proj(x, W_l, bl, W_r, br):
    grid = (N // RB,)
    return pl.pallas_call(
        _proj_body,
        grid=grid,
        in_specs=[
            pl.BlockSpec((RB, D), lambda i: (i, 0)),
            pl.BlockSpec((D, HC), lambda i: (0, 0)),
            pl.BlockSpec((1, HC), lambda i: (0, 0)),
            pl.BlockSpec((D, HC), lambda i: (0, 0)),
            pl.BlockSpec((1, HC), lambda i: (0, 0)),
        ],
        out_specs=[
            pl.BlockSpec((RB, HC), lambda i: (i, 0)),
            pl.BlockSpec((RB, HC), lambda i: (i, 0)),
            pl.BlockSpec((RB, PK), lambda i: (i, 0)),
            pl.BlockSpec((RB, PK), lambda i: (i, 0)),
        ],
        out_shape=[
            jax.ShapeDtypeStruct((N, HC), jnp.float32),
            jax.ShapeDtypeStruct((N, HC), jnp.float32),
            jax.ShapeDtypeStruct((N, PK), jnp.float32),
            jax.ShapeDtypeStruct((N, PK), jnp.float32),
        ],
    )(x, W_l, bl, W_r, br)


_sc_mesh = plsc.VectorSubcoreMesh(
    core_axis_name="c", subcore_axis_name="s", num_cores=NC, num_subcores=NS)


@functools.partial(
    pl.kernel,
    out_type=[
        jax.ShapeDtypeStruct((E, PK), jnp.float32),
        jax.ShapeDtypeStruct((E, PK), jnp.float32),
    ],
    mesh=_sc_mesh,
    scratch_types=[
        pltpu.VMEM((IRWG, KG), jnp.int32),
        pltpu.VMEM((IRWG, KG), jnp.int32),
        pltpu.VMEM((EG, PK), jnp.float32),
        pltpu.VMEM((EG, PK), jnp.float32),
        pltpu.VMEM((EG, PK), jnp.float32),
        pltpu.VMEM((EG, PK), jnp.float32),
        pltpu.SemaphoreType.DMA,
        pltpu.SemaphoreType.DMA,
        pltpu.SemaphoreType.DMA,
    ],
    compiler_params=pltpu.CompilerParams(use_tc_tiling_on_sc=False),
)
def _sc_gather(xl_hbm, xr_hbm, src2_hbm, dst2_hbm, xlg_out, xrg_out,
               sidx, didx, xlb0, xrb0, xlb1, xrb1, semA, semB, semW):
    c = lax.axis_index("c")
    s = lax.axis_index("s")
    wid = s * NC + c
    base_row = wid * IRWG
    edge0 = wid * EPW
    pltpu.sync_copy(src2_hbm.at[pl.ds(base_row, IRWG)], sidx)
    pltpu.sync_copy(dst2_hbm.at[pl.ds(base_row, IRWG)], didx)

    def _fire(row, xlb, xrb, sem):
        hs = []
        for g in range(GG):
            hs.append(pltpu.async_copy(
                xl_hbm.at[sidx.at[row + g]], xlb.at[pl.ds(g * KG, KG)], sem))
            hs.append(pltpu.async_copy(
                xr_hbm.at[didx.at[row + g]], xrb.at[pl.ds(g * KG, KG)], sem))
        return hs

    def body(t, _):
        rowa = 2 * t * GG
        e0a = edge0 + 2 * t * EG
        hsA = _fire(rowa, xlb0, xrb0, semA)
        for h in hsA:
            h.wait()
        hw0 = pltpu.async_copy(xlb0, xlg_out.at[pl.ds(e0a, EG)], semW)
        hw1 = pltpu.async_copy(xrb0, xrg_out.at[pl.ds(e0a, EG)], semW)
        hsB = _fire(rowa + GG, xlb1, xrb1, semB)
        for h in hsB:
            h.wait()
        hw0.wait()
        hw1.wait()
        pltpu.sync_copy(xlb1, xlg_out.at[pl.ds(e0a + EG, EG)])
        pltpu.sync_copy(xrb1, xrg_out.at[pl.ds(e0a + EG, EG)])
        return 0

    lax.fori_loop(0, NBG, body, 0)


def _edge_w_body(ea_ref, xlg_ref, xrg_ref, we_ref, be_ref, attf_ref,
                 w_ref, den_ref, cs_ref):
    i = pl.program_id(0)
    ea = ea_ref[...]
    e = jnp.dot(ea, we_ref[...], preferred_element_type=jnp.float32) + be_ref[...]
    xl_lo, xl_hi = _unpack_bf16_halves(xlg_ref[...])
    xr_lo, xr_hi = _unpack_bf16_halves(xrg_ref[...])
    attf = attf_ref[...]
    m_lo = xl_lo + xr_lo + e[:, :PK]
    m_hi = xl_hi + xr_hi + e[:, PK:]
    ga_lo = jnp.where(m_lo > 0.0, m_lo, 0.2 * m_lo) * attf[:, :PK]
    ga_hi = jnp.where(m_hi > 0.0, m_hi, 0.2 * m_hi) * attf[:, PK:]
    hsel = _hsel_p()
    logits = (jnp.dot(ga_lo, hsel[:PK], preferred_element_type=jnp.float32)
              + jnp.dot(ga_hi, hsel[PK:], preferred_element_type=jnp.float32))
    ex = jnp.exp(logits)
    exb_lo = jnp.dot(ex, hsel[:PK].T, preferred_element_type=jnp.float32)
    exb_hi = jnp.dot(ex, hsel[PK:].T, preferred_element_type=jnp.float32)
    w_ref[:, :PK] = xl_lo * exb_lo
    w_ref[:, PK:] = xl_hi * exb_hi
    dmap = (lax.broadcasted_iota(jnp.int32, (H, DEN), 1)
            == 4 * lax.broadcasted_iota(jnp.int32, (H, DEN), 0)).astype(jnp.float32)
    den_ref[...] = jnp.dot(ex, dmap, preferred_element_type=jnp.float32)

    @pl.when(i == 0)
    def _():
        cs_ref[...] = jnp.zeros_like(cs_ref)

    cs_ref[...] += jnp.sum(ea, axis=0, keepdims=True)


def _edge_w(ea, xlg, xrg, W_e, be, attf):
    grid = (E // EB,)
    return pl.pallas_call(
        _edge_w_body,
        grid=grid,
        in_specs=[
            pl.BlockSpec((EB, DE), lambda i: (i, 0)),
            pl.BlockSpec((EB, PK), lambda i: (i, 0)),
            pl.BlockSpec((EB, PK), lambda i: (i, 0)),
            pl.BlockSpec((DE, HC), lambda i: (0, 0)),
            pl.BlockSpec((1, HC), lambda i: (0, 0)),
            pl.BlockSpec((1, HC), lambda i: (0, 0)),
        ],
        out_specs=[
            pl.BlockSpec((EB, HC), lambda i: (i, 0)),
            pl.BlockSpec((EB, DEN), lambda i: (i, 0)),
            pl.BlockSpec((1, DE), lambda i: (0, 0)),
        ],
        out_shape=[
            jax.ShapeDtypeStruct((E, HC), jnp.float32),
            jax.ShapeDtypeStruct((E, DEN), jnp.float32),
            jax.ShapeDtypeStruct((1, DE), jnp.float32),
        ],
    )(ea, xlg, xrg, W_e, be, attf)


@functools.partial(
    pl.kernel,
    out_type=[
        jax.ShapeDtypeStruct((NC, NP, HC), jnp.float32),
        jax.ShapeDtypeStruct((NC, NP, DEN), jnp.float32),
    ],
    mesh=_sc_mesh,
    scratch_types=[
        pltpu.VMEM((NCH, KS), jnp.int32),
        pltpu.VMEM((2 * KS, HC), jnp.float32),
        pltpu.VMEM((2 * KS, DEN), jnp.float32),
        pltpu.VMEM((ZR, HC), jnp.float32),
        pltpu.VMEM((ZR, DEN), jnp.float32),
        pltpu.VMEM_SHARED((NP, HC), jnp.float32),
        pltpu.VMEM_SHARED((NP, DEN), jnp.float32),
        pltpu.SemaphoreType.DMA,
        pltpu.SemaphoreType.DMA,
        pltpu.SemaphoreType.DMA,
    ],
    compiler_params=pltpu.CompilerParams(use_tc_tiling_on_sc=False),
)
def _sc_scatter(w_hbm, den_hbm, dst2_hbm, acc_out, den_out,
                didx, wbuf, dbuf, zrow_v, zden_v, acc_sh, den_sh,
                sem0, sem1, semS):
    c = lax.axis_index("c")
    s = lax.axis_index("s")
    wid = s * NC + c
    zeros16 = jnp.zeros((16,), jnp.float32)

    def zrow_body(i, _):
        for v in range(HC // 16):
            zrow_v[i, pl.ds(v * 16, 16)] = zeros16
        return 0

    lax.fori_loop(0, ZR, zrow_body, 0)

    def zden_body(i, _):
        zden_v[i, :] = zeros16
        return 0

    lax.fori_loop(0, ZR, zden_body, 0)

    row0 = s * RPT

    def zinit_body(t, _):
        pltpu.sync_copy(zrow_v, acc_sh.at[pl.ds(row0 + t * ZR, ZR), :])
        pltpu.sync_copy(zden_v, den_sh.at[pl.ds(row0 + t * ZR, ZR), :])
        return 0

    lax.fori_loop(0, RPT // ZR, zinit_body, 0)
    plsc.subcore_barrier()

    base_row = wid * NCH
    edge0 = wid * EPW
    pltpu.sync_copy(dst2_hbm.at[pl.ds(base_row, NCH)], didx)

    def body(t, _):
        row = t * 2
        e0 = edge0 + t * 2 * KS
        h0 = pltpu.async_copy(w_hbm.at[pl.ds(e0, KS)], wbuf.at[pl.ds(0, KS)], sem0)
        h1 = pltpu.async_copy(den_hbm.at[pl.ds(e0, KS)], dbuf.at[pl.ds(0, KS)], sem0)
        h2 = pltpu.async_copy(w_hbm.at[pl.ds(e0 + KS, KS)], wbuf.at[pl.ds(KS, KS)], sem1)
        h3 = pltpu.async_copy(den_hbm.at[pl.ds(e0 + KS, KS)], dbuf.at[pl.ds(KS, KS)], sem1)
        h0.wait()
        h1.wait()
        s0 = pltpu.async_copy(wbuf.at[pl.ds(0, KS)], acc_sh.at[didx.at[row]], semS, add=True)
        s1 = pltpu.async_copy(dbuf.at[pl.ds(0, KS)], den_sh.at[didx.at[row]], semS, add=True)
        h2.wait()
        h3.wait()
        s2 = pltpu.async_copy(wbuf.at[pl.ds(KS, KS)], acc_sh.at[didx.at[row + 1]], semS, add=True)
        s3 = pltpu.async_copy(dbuf.at[pl.ds(KS, KS)], den_sh.at[didx.at[row + 1]], semS, add=True)
        s0.wait()
        s1.wait()
        s2.wait()
        s3.wait()
        return 0

    lax.fori_loop(0, PAIRS, body, 0)

    e0 = edge0 + (NCH - 1) * KS
    pltpu.sync_copy(w_hbm.at[pl.ds(e0, KS)], wbuf.at[pl.ds(0, KS)])
    pltpu.sync_copy(den_hbm.at[pl.ds(e0, KS)], dbuf.at[pl.ds(0, KS)])
    pltpu.sync_copy(wbuf.at[pl.ds(0, KS)], acc_sh.at[didx.at[NCH - 1]], add=True)
    pltpu.sync_copy(dbuf.at[pl.ds(0, KS)], den_sh.at[didx.at[NCH - 1]], add=True)

    plsc.subcore_barrier()
    pltpu.sync_copy(acc_sh.at[pl.ds(row0, RPT), :], acc_out.at[c, pl.ds(row0, RPT), :])
    pltpu.sync_copy(den_sh.at[pl.ds(row0, RPT), :], den_out.at[c, pl.ds(row0, RPT), :])


def _merge_body(xl_ref, xr_ref, acc_ref, den_ref, cs_ref, we_ref, be_ref,
                attf_ref, bias_ref, out_ref, st_ref):
    i = pl.program_id(0)
    hsel = _hsel_p()
    eloop = jnp.dot(cs_ref[...] * (1.0 / E), we_ref[...],
                    preferred_element_type=jnp.float32) + be_ref[...]
    xl = xl_ref[...]
    m = xl + xr_ref[...] + eloop
    ga = jnp.where(m > 0.0, m, 0.2 * m) * attf_ref[...]
    logits = jnp.dot(ga, hsel, preferred_element_type=jnp.float32)
    ex = jnp.exp(logits)
    exb = jnp.dot(ex, hsel.T, preferred_element_type=jnp.float32)
    num = acc_ref[0] + acc_ref[1] + exb * xl
    dsel = (lax.broadcasted_iota(jnp.int32, (DEN, H), 0)
            == 4 * lax.broadcasted_iota(jnp.int32, (DEN, H), 1)).astype(jnp.float32)
    den4 = jnp.dot(den_ref[0] + den_ref[1], dsel,
                   preferred_element_type=jnp.float32) + ex
    denb = jnp.dot(den4, hsel.T, preferred_element_type=jnp.float32)
    out = num / (denb + 1e-16) + bias_ref[...]
    out_ref[...] = out

    @pl.when(i == 0)
    def _():
        st_ref[...] = jnp.zeros_like(st_ref)

    st_ref[0:1, :] += jnp.sum(out, axis=0, keepdims=True)
    st_ref[1:2, :] += jnp.sum(out * out, axis=0, keepdims=True)


def _merge(xl, xr, acc, den, cs, W_e, be, attf, bias):
    grid = (N // RB,)
    return pl.pallas_call(
        _merge_body,
        grid=grid,
        in_specs=[
            pl.BlockSpec((RB, HC), lambda i: (i, 0)),
            pl.BlockSpec((RB, HC), lambda i: (i, 0)),
            pl.BlockSpec((NC, RB, HC), lambda i: (0, i, 0)),
            pl.BlockSpec((NC, RB, DEN), lambda i: (0, i, 0)),
            pl.BlockSpec((1, DE), lambda i: (0, 0)),
            pl.BlockSpec((DE, HC), lambda i: (0, 0)),
            pl.BlockSpec((1, HC), lambda i: (0, 0)),
            pl.BlockSpec((1, HC), lambda i: (0, 0)),
            pl.BlockSpec((1, HC), lambda i: (0, 0)),
        ],
        out_specs=[
            pl.BlockSpec((RB, HC), lambda i: (i, 0)),
            pl.BlockSpec((2, HC), lambda i: (0, 0)),
        ],
        out_shape=[
            jax.ShapeDtypeStruct((N, HC), jnp.float32),
            jax.ShapeDtypeStruct((2, HC), jnp.float32),
        ],
    )(xl, xr, acc, den, cs, W_e, be, attf, bias)


def _norm_body(op_ref, st_ref, gw_ref, gb_ref, gm_ref, o_ref):
    mean = st_ref[0:1, :] * (1.0 / N)
    msq = st_ref[1:2, :] * (1.0 / N)
    gm = gm_ref[...]
    var = msq - gm * mean * mean * (2.0 - gm)
    inv = lax.rsqrt(var + 1e-5)
    o = gw_ref[...] * (op_ref[...] - gm * mean) * inv + gb_ref[...]
    o = jnp.maximum(o, 0.0)
    j = lax.broadcasted_iota(jnp.int32, (HC, HC), 0)
    k = lax.broadcasted_iota(jnp.int32, (HC, HC), 1)
    pm = (k == _perm_feature(j)).astype(jnp.float32)
    o_ref[...] = jnp.dot(o, pm, preferred_element_type=jnp.float32)


def _norm(out_pre, st, gw, gb, gm):
    grid = (N // RB,)
    return pl.pallas_call(
        _norm_body,
        grid=grid,
        in_specs=[
            pl.BlockSpec((RB, HC), lambda i: (i, 0)),
            pl.BlockSpec((2, HC), lambda i: (0, 0)),
            pl.BlockSpec((1, HC), lambda i: (0, 0)),
            pl.BlockSpec((1, HC), lambda i: (0, 0)),
            pl.BlockSpec((1, HC), lambda i: (0, 0)),
        ],
        out_specs=pl.BlockSpec((RB, HC), lambda i: (i, 0)),
        out_shape=jax.ShapeDtypeStruct((N, HC), jnp.float32),
    )(out_pre, st, gw, gb, gm)


def kernel(x, edge_index, edge_attr, W_l, b_l, W_r, b_r, W_e, b_e, att,
           bias_out, gn_weight, gn_bias, gn_mean_scale):
    src = edge_index[0]
    dst = edge_index[1]
    perm = jnp.concatenate([jnp.arange(0, HC, 2, dtype=jnp.int32),
                            jnp.arange(1, HC, 2, dtype=jnp.int32)])
    bl = b_l[perm].reshape(1, HC)
    br = b_r[perm].reshape(1, HC)
    be = b_e[perm].reshape(1, HC)
    attf = att.reshape(HC)[perm].reshape(1, HC)
    bias = bias_out[perm].reshape(1, HC)
    gw = gn_weight[perm].reshape(1, HC)
    gb = gn_bias[perm].reshape(1, HC)
    gm = gn_mean_scale[perm].reshape(1, HC)
    W_lp = W_l[:, perm]
    W_rp = W_r[:, perm]
    W_ep = W_e[:, perm]

    xl, xr, xlp, xrp = _proj(x, W_lp, bl, W_rp, br)
    xlg, xrg = _sc_gather(xlp, xrp, src.reshape(E // KG, KG),
                          dst.reshape(E // KG, KG))
    w, den_e, cs = _edge_w(edge_attr, xlg, xrg, W_ep, be, attf)
    acc, den = _sc_scatter(w, den_e, dst.reshape(E // KS, KS))

    out_pre, st = _merge(xl, xr, acc, den, cs, W_ep, be, attf, bias)
    return _norm(out_pre, st, gw, gb, gm)


# speedup vs baseline: 1.1979x; 1.1979x over previous
import functools

import jax
import jax.numpy as jnp
from jax import lax
from jax.experimental import pallas as pl
from jax.experimental.pallas import tpu as pltpu
from jax.experimental.pallas import tpu_sc as plsc

N = 10000
E = 320000
D = 128
DE = 16
H = 4
C = 32
HC = H * C
DEN = 16

NC = 2
NS = 16
NW = NC * NS
EPW = E // NW
NP = 10240
RPT = NP // NS
ZR = 16

KG = 40
GG = 5
EG = GG * KG
NBG = EPW // (2 * EG)
IRWG = E // KG // NW

KS = 80
NCH = EPW // KS
PAIRS = NCH // 2

RB = 2000
EB = 4000


def _proj_body(x_ref, wl_ref, bl_ref, wr_ref, br_ref, xl_ref, xr_ref):
    x = x_ref[...]
    xl_ref[...] = jnp.dot(x, wl_ref[...], preferred_element_type=jnp.float32) + bl_ref[...]
    xr_ref[...] = jnp.dot(x, wr_ref[...], preferred_element_type=jnp.float32) + br_ref[...]


def _proj(x, W_l, bl, W_r, br):
    grid = (N // RB,)
    return pl.pallas_call(
        _proj_body,
        grid=grid,
        in_specs=[
            pl.BlockSpec((RB, D), lambda i: (i, 0)),
            pl.BlockSpec((D, HC), lambda i: (0, 0)),
            pl.BlockSpec((1, HC), lambda i: (0, 0)),
            pl.BlockSpec((D, HC), lambda i: (0, 0)),
            pl.BlockSpec((1, HC), lambda i: (0, 0)),
        ],
        out_specs=[
            pl.BlockSpec((RB, HC), lambda i: (i, 0)),
            pl.BlockSpec((RB, HC), lambda i: (i, 0)),
        ],
        out_shape=[
            jax.ShapeDtypeStruct((N, HC), jnp.float32),
            jax.ShapeDtypeStruct((N, HC), jnp.float32),
        ],
    )(x, W_l, bl, W_r, br)


_sc_mesh = plsc.VectorSubcoreMesh(
    core_axis_name="c", subcore_axis_name="s", num_cores=NC, num_subcores=NS)


@functools.partial(
    pl.kernel,
    out_type=[
        jax.ShapeDtypeStruct((E, HC), jnp.float32),
        jax.ShapeDtypeStruct((E, HC), jnp.float32),
    ],
    mesh=_sc_mesh,
    scratch_types=[
        pltpu.VMEM((IRWG, KG), jnp.int32),
        pltpu.VMEM((IRWG, KG), jnp.int32),
        pltpu.VMEM((EG, HC), jnp.float32),
        pltpu.VMEM((EG, HC), jnp.float32),
        pltpu.VMEM((EG, HC), jnp.float32),
        pltpu.VMEM((EG, HC), jnp.float32),
        pltpu.SemaphoreType.DMA,
        pltpu.SemaphoreType.DMA,
        pltpu.SemaphoreType.DMA,
    ],
    compiler_params=pltpu.CompilerParams(use_tc_tiling_on_sc=False),
)
def _sc_gather(xl_hbm, xr_hbm, src2_hbm, dst2_hbm, xlg_out, xrg_out,
               sidx, didx, xlb0, xrb0, xlb1, xrb1, semA, semB, semW):
    c = lax.axis_index("c")
    s = lax.axis_index("s")
    wid = s * NC + c
    base_row = wid * IRWG
    edge0 = wid * EPW
    pltpu.sync_copy(src2_hbm.at[pl.ds(base_row, IRWG)], sidx)
    pltpu.sync_copy(dst2_hbm.at[pl.ds(base_row, IRWG)], didx)

    def _fire(row, xlb, xrb, sem):
        hs = []
        for g in range(GG):
            hs.append(pltpu.async_copy(
                xl_hbm.at[sidx.at[row + g]], xlb.at[pl.ds(g * KG, KG)], sem))
            hs.append(pltpu.async_copy(
                xr_hbm.at[didx.at[row + g]], xrb.at[pl.ds(g * KG, KG)], sem))
        return hs

    def body(t, _):
        rowa = 2 * t * GG
        e0a = edge0 + 2 * t * EG
        hsA = _fire(rowa, xlb0, xrb0, semA)
        for h in hsA:
            h.wait()
        hw0 = pltpu.async_copy(xlb0, xlg_out.at[pl.ds(e0a, EG)], semW)
        hw1 = pltpu.async_copy(xrb0, xrg_out.at[pl.ds(e0a, EG)], semW)
        hsB = _fire(rowa + GG, xlb1, xrb1, semB)
        for h in hsB:
            h.wait()
        hw0.wait()
        hw1.wait()
        pltpu.sync_copy(xlb1, xlg_out.at[pl.ds(e0a + EG, EG)])
        pltpu.sync_copy(xrb1, xrg_out.at[pl.ds(e0a + EG, EG)])
        return 0

    lax.fori_loop(0, NBG, body, 0)


def _edge_w_body(ea_ref, xlg_ref, xrg_ref, we_ref, be_ref, attf_ref,
                 w_ref, den_ref, cs_ref):
    i = pl.program_id(0)
    ea = ea_ref[...]
    e = jnp.dot(ea, we_ref[...], preferred_element_type=jnp.float32) + be_ref[...]
    xlg = xlg_ref[...]
    m = xlg + xrg_ref[...] + e
    ga = jnp.where(m > 0.0, m, 0.2 * m) * attf_ref[...]
    hsel = (lax.broadcasted_iota(jnp.int32, (HC, H), 0) // C
            == lax.broadcasted_iota(jnp.int32, (HC, H), 1)).astype(jnp.float32)
    logits = jnp.dot(ga, hsel, preferred_element_type=jnp.float32)
    ex = jnp.exp(logits)
    exb = jnp.dot(ex, hsel.T, preferred_element_type=jnp.float32)
    w_ref[...] = xlg * exb
    dmap = (lax.broadcasted_iota(jnp.int32, (H, DEN), 1)
            == 4 * lax.broadcasted_iota(jnp.int32, (H, DEN), 0)).astype(jnp.float32)
    den_ref[...] = jnp.dot(ex, dmap, preferred_element_type=jnp.float32)

    @pl.when(i == 0)
    def _():
        cs_ref[...] = jnp.zeros_like(cs_ref)

    cs_ref[...] += jnp.sum(ea, axis=0, keepdims=True)


def _edge_w(ea, xlg, xrg, W_e, be, attf):
    grid = (E // EB,)
    return pl.pallas_call(
        _edge_w_body,
        grid=grid,
        in_specs=[
            pl.BlockSpec((EB, DE), lambda i: (i, 0)),
            pl.BlockSpec((EB, HC), lambda i: (i, 0)),
            pl.BlockSpec((EB, HC), lambda i: (i, 0)),
            pl.BlockSpec((DE, HC), lambda i: (0, 0)),
            pl.BlockSpec((1, HC), lambda i: (0, 0)),
            pl.BlockSpec((1, HC), lambda i: (0, 0)),
        ],
        out_specs=[
            pl.BlockSpec((EB, HC), lambda i: (i, 0)),
            pl.BlockSpec((EB, DEN), lambda i: (i, 0)),
            pl.BlockSpec((1, DE), lambda i: (0, 0)),
        ],
        out_shape=[
            jax.ShapeDtypeStruct((E, HC), jnp.float32),
            jax.ShapeDtypeStruct((E, DEN), jnp.float32),
            jax.ShapeDtypeStruct((1, DE), jnp.float32),
        ],
    )(ea, xlg, xrg, W_e, be, attf)


@functools.partial(
    pl.kernel,
    out_type=[
        jax.ShapeDtypeStruct((NC, NP, HC), jnp.float32),
        jax.ShapeDtypeStruct((NC, NP, DEN), jnp.float32),
    ],
    mesh=_sc_mesh,
    scratch_types=[
        pltpu.VMEM((NCH, KS), jnp.int32),
        pltpu.VMEM((2 * KS, HC), jnp.float32),
        pltpu.VMEM((2 * KS, DEN), jnp.float32),
        pltpu.VMEM((ZR, HC), jnp.float32),
        pltpu.VMEM((ZR, DEN), jnp.float32),
        pltpu.VMEM_SHARED((NP, HC), jnp.float32),
        pltpu.VMEM_SHARED((NP, DEN), jnp.float32),
        pltpu.SemaphoreType.DMA,
        pltpu.SemaphoreType.DMA,
        pltpu.SemaphoreType.DMA,
    ],
    compiler_params=pltpu.CompilerParams(use_tc_tiling_on_sc=False),
)
def _sc_scatter(w_hbm, den_hbm, dst2_hbm, acc_out, den_out,
                didx, wbuf, dbuf, zrow_v, zden_v, acc_sh, den_sh,
                sem0, sem1, semS):
    c = lax.axis_index("c")
    s = lax.axis_index("s")
    wid = s * NC + c
    zeros16 = jnp.zeros((16,), jnp.float32)

    def zrow_body(i, _):
        for v in range(HC // 16):
            zrow_v[i, pl.ds(v * 16, 16)] = zeros16
        return 0

    lax.fori_loop(0, ZR, zrow_body, 0)

    def zden_body(i, _):
        zden_v[i, :] = zeros16
        return 0

    lax.fori_loop(0, ZR, zden_body, 0)

    row0 = s * RPT

    def zinit_body(t, _):
        pltpu.sync_copy(zrow_v, acc_sh.at[pl.ds(row0 + t * ZR, ZR), :])
        pltpu.sync_copy(zden_v, den_sh.at[pl.ds(row0 + t * ZR, ZR), :])
        return 0

    lax.fori_loop(0, RPT // ZR, zinit_body, 0)
    plsc.subcore_barrier()

    base_row = wid * NCH
    edge0 = wid * EPW
    pltpu.sync_copy(dst2_hbm.at[pl.ds(base_row, NCH)], didx)

    def body(t, _):
        row = t * 2
        e0 = edge0 + t * 2 * KS
        h0 = pltpu.async_copy(w_hbm.at[pl.ds(e0, KS)], wbuf.at[pl.ds(0, KS)], sem0)
        h1 = pltpu.async_copy(den_hbm.at[pl.ds(e0, KS)], dbuf.at[pl.ds(0, KS)], sem0)
        h2 = pltpu.async_copy(w_hbm.at[pl.ds(e0 + KS, KS)], wbuf.at[pl.ds(KS, KS)], sem1)
        h3 = pltpu.async_copy(den_hbm.at[pl.ds(e0 + KS, KS)], dbuf.at[pl.ds(KS, KS)], sem1)
        h0.wait()
        h1.wait()
        s0 = pltpu.async_copy(wbuf.at[pl.ds(0, KS)], acc_sh.at[didx.at[row]], semS, add=True)
        s1 = pltpu.async_copy(dbuf.at[pl.ds(0, KS)], den_sh.at[didx.at[row]], semS, add=True)
        h2.wait()
        h3.wait()
        s2 = pltpu.async_copy(wbuf.at[pl.ds(KS, KS)], acc_sh.at[didx.at[row + 1]], semS, add=True)
        s3 = pltpu.async_copy(dbuf.at[pl.ds(KS, KS)], den_sh.at[didx.at[row + 1]], semS, add=True)
        s0.wait()
        s1.wait()
        s2.wait()
        s3.wait()
        return 0

    lax.fori_loop(0, PAIRS, body, 0)

    e0 = edge0 + (NCH - 1) * KS
    pltpu.sync_copy(w_hbm.at[pl.ds(e0, KS)], wbuf.at[pl.ds(0, KS)])
    pltpu.sync_copy(den_hbm.at[pl.ds(e0, KS)], dbuf.at[pl.ds(0, KS)])
    pltpu.sync_copy(wbuf.at[pl.ds(0, KS)], acc_sh.at[didx.at[NCH - 1]], add=True)
    pltpu.sync_copy(dbuf.at[pl.ds(0, KS)], den_sh.at[didx.at[NCH - 1]], add=True)

    plsc.subcore_barrier()
    pltpu.sync_copy(acc_sh.at[pl.ds(row0, RPT), :], acc_out.at[c, pl.ds(row0, RPT), :])
    pltpu.sync_copy(den_sh.at[pl.ds(row0, RPT), :], den_out.at[c, pl.ds(row0, RPT), :])


def _merge_body(xl_ref, xr_ref, acc_ref, den_ref, cs_ref, we_ref, be_ref,
                attf_ref, bias_ref, out_ref, st_ref):
    i = pl.program_id(0)
    hsel = (lax.broadcasted_iota(jnp.int32, (HC, H), 0) // C
            == lax.broadcasted_iota(jnp.int32, (HC, H), 1)).astype(jnp.float32)
    eloop = jnp.dot(cs_ref[...] * (1.0 / E), we_ref[...],
                    preferred_element_type=jnp.float32) + be_ref[...]
    xl = xl_ref[...]
    m = xl + xr_ref[...] + eloop
    ga = jnp.where(m > 0.0, m, 0.2 * m) * attf_ref[...]
    logits = jnp.dot(ga, hsel, preferred_element_type=jnp.float32)
    ex = jnp.exp(logits)
    exb = jnp.dot(ex, hsel.T, preferred_element_type=jnp.float32)
    num = acc_ref[0] + acc_ref[1] + exb * xl
    dsel = (lax.broadcasted_iota(jnp.int32, (DEN, H), 0)
            == 4 * lax.broadcasted_iota(jnp.int32, (DEN, H), 1)).astype(jnp.float32)
    den4 = jnp.dot(den_ref[0] + den_ref[1], dsel,
                   preferred_element_type=jnp.float32) + ex
    denb = jnp.dot(den4, hsel.T, preferred_element_type=jnp.float32)
    out = num / (denb + 1e-16) + bias_ref[...]
    out_ref[...] = out

    @pl.when(i == 0)
    def _():
        st_ref[...] = jnp.zeros_like(st_ref)

    st_ref[0:1, :] += jnp.sum(out, axis=0, keepdims=True)
    st_ref[1:2, :] += jnp.sum(out * out, axis=0, keepdims=True)


def _merge(xl, xr, acc, den, cs, W_e, be, attf, bias):
    grid = (N // RB,)
    return pl.pallas_call(
        _merge_body,
        grid=grid,
        in_specs=[
            pl.BlockSpec((RB, HC), lambda i: (i, 0)),
            pl.BlockSpec((RB, HC), lambda i: (i, 0)),
            pl.BlockSpec((NC, RB, HC), lambda i: (0, i, 0)),
            pl.BlockSpec((NC, RB, DEN), lambda i: (0, i, 0)),
            pl.BlockSpec((1, DE), lambda i: (0, 0)),
            pl.BlockSpec((DE, HC), lambda i: (0, 0)),
            pl.BlockSpec((1, HC), lambda i: (0, 0)),
            pl.BlockSpec((1, HC), lambda i: (0, 0)),
            pl.BlockSpec((1, HC), lambda i: (0, 0)),
        ],
        out_specs=[
            pl.BlockSpec((RB, HC), lambda i: (i, 0)),
            pl.BlockSpec((2, HC), lambda i: (0, 0)),
        ],
        out_shape=[
            jax.ShapeDtypeStruct((N, HC), jnp.float32),
            jax.ShapeDtypeStruct((2, HC), jnp.float32),
        ],
    )(xl, xr, acc, den, cs, W_e, be, attf, bias)


def _norm_body(op_ref, st_ref, gw_ref, gb_ref, gm_ref, o_ref):
    mean = st_ref[0:1, :] * (1.0 / N)
    msq = st_ref[1:2, :] * (1.0 / N)
    gm = gm_ref[...]
    var = msq - gm * mean * mean * (2.0 - gm)
    inv = lax.rsqrt(var + 1e-5)
    o = gw_ref[...] * (op_ref[...] - gm * mean) * inv + gb_ref[...]
    o_ref[...] = jnp.maximum(o, 0.0)


def _norm(out_pre, st, gw, gb, gm):
    grid = (N // RB,)
    return pl.pallas_call(
        _norm_body,
        grid=grid,
        in_specs=[
            pl.BlockSpec((RB, HC), lambda i: (i, 0)),
            pl.BlockSpec((2, HC), lambda i: (0, 0)),
            pl.BlockSpec((1, HC), lambda i: (0, 0)),
            pl.BlockSpec((1, HC), lambda i: (0, 0)),
            pl.BlockSpec((1, HC), lambda i: (0, 0)),
        ],
        out_specs=pl.BlockSpec((RB, HC), lambda i: (i, 0)),
        out_shape=jax.ShapeDtypeStruct((N, HC), jnp.float32),
    )(out_pre, st, gw, gb, gm)


def kernel(x, edge_index, edge_attr, W_l, b_l, W_r, b_r, W_e, b_e, att,
           bias_out, gn_weight, gn_bias, gn_mean_scale):
    src = edge_index[0]
    dst = edge_index[1]
    bl = b_l.reshape(1, HC)
    br = b_r.reshape(1, HC)
    be = b_e.reshape(1, HC)
    attf = att.reshape(1, HC)
    bias = bias_out.reshape(1, HC)
    gw = gn_weight.reshape(1, HC)
    gb = gn_bias.reshape(1, HC)
    gm = gn_mean_scale.reshape(1, HC)

    xl, xr = _proj(x, W_l, bl, W_r, br)
    xlg, xrg = _sc_gather(xl, xr, src.reshape(E // KG, KG),
                          dst.reshape(E // KG, KG))
    w, den_e, cs = _edge_w(edge_attr, xlg, xrg, W_e, be, attf)
    acc, den = _sc_scatter(w, den_e, dst.reshape(E // KS, KS))

    out_pre, st = _merge(xl, xr, acc, den, cs, W_e, be, attf, bias)
    return _norm(out_pre, st, gw, gb, gm)
